# baseline (speedup 1.0000x reference)
import jax
import jax.numpy as jnp
from jax.experimental import pallas as pl
from jax.experimental.pallas import tpu as pltpu


def _body(ac_ref, at_ref, x_ref, w_ref, bias_ref, ct_ref, tt_ref, o_ref):
    xt = x_ref[0, 0]
    acc = jax.lax.dot_general(
        xt, w_ref[...], (((1,), (0,)), ((), ())),
        preferred_element_type=jnp.float32,
    )
    res = (acc + bias_ref[...]
           + ac_ref[0, 0] * ct_ref[0]
           + at_ref[0, 0] * tt_ref[...])
    o_ref[0] = res


def kernel(x, W, bias, chan_table, time_table, alpha_c, alpha_t):
    b, ch, t_len, in_dim = x.shape
    d = W.shape[0]
    wt = W.T
    bias2 = bias.reshape(1, d)
    ct3 = chan_table.reshape(ch, 1, d)
    ac = alpha_c.reshape(1, 1)
    at = alpha_t.reshape(1, 1)

    out = pl.pallas_call(
        _body,
        grid=(b, ch),
        in_specs=[
            pl.BlockSpec(memory_space=pltpu.SMEM),
            pl.BlockSpec(memory_space=pltpu.SMEM),
            pl.BlockSpec((1, 1, t_len, in_dim), lambda i, j: (i, j, 0, 0)),
            pl.BlockSpec((in_dim, d), lambda i, j: (0, 0)),
            pl.BlockSpec((1, d), lambda i, j: (0, 0)),
            pl.BlockSpec((1, 1, d), lambda i, j: (j, 0, 0)),
            pl.BlockSpec((t_len, d), lambda i, j: (0, 0)),
        ],
        out_specs=pl.BlockSpec((1, t_len, d), lambda i, j: (i, 0, j)),
        out_shape=jax.ShapeDtypeStruct((b, t_len, ch * d), jnp.float32),
        compiler_params=pltpu.CompilerParams(
            dimension_semantics=("parallel", "parallel"),
        ),
    )(ac, at, x, wt, bias2, ct3, time_table)
    return out.reshape(b, t_len * ch, d)


# speedup vs baseline: 1.0066x; 1.0066x over previous
import jax
import jax.numpy as jnp
from jax.experimental import pallas as pl
from jax.experimental.pallas import tpu as pltpu


def _body(x_ref, w_ref, ct_ref, tt_ref, o_ref):
    xt = x_ref[0, 0]
    acc = jax.lax.dot_general(
        xt, w_ref[...], (((1,), (0,)), ((), ())),
        preferred_element_type=jnp.float32,
    )
    o_ref[0] = acc + tt_ref[...] + ct_ref[0]


def kernel(x, W, bias, chan_table, time_table, alpha_c, alpha_t):
    b, ch, t_len, in_dim = x.shape
    d = W.shape[0]
    wt = W.T
    ct3 = (alpha_c * chan_table).reshape(ch, 1, d)
    tvec = bias[None, :] + alpha_t * time_table

    out = pl.pallas_call(
        _body,
        grid=(b, ch),
        in_specs=[
            pl.BlockSpec((1, 1, t_len, in_dim), lambda i, j: (i, j, 0, 0)),
            pl.BlockSpec((in_dim, d), lambda i, j: (0, 0)),
            pl.BlockSpec((1, 1, d), lambda i, j: (j, 0, 0)),
            pl.BlockSpec((t_len, d), lambda i, j: (0, 0)),
        ],
        out_specs=pl.BlockSpec((1, t_len, d), lambda i, j: (i, 0, j)),
        out_shape=jax.ShapeDtypeStruct((b, t_len, ch * d), jnp.float32),
        compiler_params=pltpu.CompilerParams(
            dimension_semantics=("parallel", "parallel"),
        ),
    )(x, wt, ct3, tvec)
    return out.reshape(b, t_len * ch, d)
